# Initial kernel scaffold; baseline (speedup 1.0000x reference)
import jax
import jax.numpy as jnp
from jax.experimental import pallas as pl


def kernel(x_bug, x_dev, ei_bug_to_dev, ei_dev_to_bug, ei_bug_dup_bug, params):
    raise NotImplementedError("write your pallas kernel here")


# speedup vs baseline: 5.0024x; 5.0024x over previous
import functools

import jax
import jax.numpy as jnp
from jax import lax
from jax.experimental import pallas as pl
from jax.experimental.pallas import tpu as pltpu
from jax.experimental.pallas import tpu_sc as plsc

N_NODE = 10000
E = 160000
BM = 400
GRID_M = N_NODE // BM


def _proj1_body(xb_ref, xd_ref, wb_ref, bb_ref, wd_ref, bd_ref, lsb_ref, lsd_ref,
                xbo_ref, xdo_ref, ab_ref, ad_ref, gb_ref, gd_ref):
    i = pl.program_id(0)
    xb = jnp.dot(xb_ref[...], wb_ref[...], preferred_element_type=jnp.float32) + bb_ref[...]
    xd = jnp.dot(xd_ref[...], wd_ref[...], preferred_element_type=jnp.float32) + bd_ref[...]
    xbo_ref[...] = xb
    xdo_ref[...] = xd
    ab = jnp.dot(xb, lsb_ref[...], preferred_element_type=jnp.float32)
    ad = jnp.dot(xd, lsd_ref[...], preferred_element_type=jnp.float32)
    ab_ref[...] = ab
    ad_ref[...] = ad
    gb = jnp.max(ab, axis=0, keepdims=True)
    gd = jnp.max(ad, axis=0, keepdims=True)

    @pl.when(i == 0)
    def _():
        gb_ref[...] = gb
        gd_ref[...] = gd

    @pl.when(i != 0)
    def _():
        gb_ref[...] = jnp.maximum(gb_ref[...], gb)
        gd_ref[...] = jnp.maximum(gd_ref[...], gd)


def _proj1(x_bug, x_dev, wb, bb, wd, bd, lsb, lsd):
    return pl.pallas_call(
        _proj1_body,
        grid=(GRID_M,),
        in_specs=[
            pl.BlockSpec((BM, 256), lambda i: (i, 0)),
            pl.BlockSpec((BM, 256), lambda i: (i, 0)),
            pl.BlockSpec((256, 256), lambda i: (0, 0)),
            pl.BlockSpec((1, 256), lambda i: (0, 0)),
            pl.BlockSpec((256, 256), lambda i: (0, 0)),
            pl.BlockSpec((1, 256), lambda i: (0, 0)),
            pl.BlockSpec((256, 32), lambda i: (0, 0)),
            pl.BlockSpec((256, 16), lambda i: (0, 0)),
        ],
        out_specs=[
            pl.BlockSpec((BM, 256), lambda i: (i, 0)),
            pl.BlockSpec((BM, 256), lambda i: (i, 0)),
            pl.BlockSpec((BM, 32), lambda i: (i, 0)),
            pl.BlockSpec((BM, 16), lambda i: (i, 0)),
            pl.BlockSpec((1, 32), lambda i: (0, 0)),
            pl.BlockSpec((1, 16), lambda i: (0, 0)),
        ],
        out_shape=[
            jax.ShapeDtypeStruct((N_NODE, 256), jnp.float32),
            jax.ShapeDtypeStruct((N_NODE, 256), jnp.float32),
            jax.ShapeDtypeStruct((N_NODE, 32), jnp.float32),
            jax.ShapeDtypeStruct((N_NODE, 16), jnp.float32),
            jax.ShapeDtypeStruct((1, 32), jnp.float32),
            jax.ShapeDtypeStruct((1, 16), jnp.float32),
        ],
    )(x_bug, x_dev, wb, bb, wd, bd, lsb, lsd)


def _norm_t_body(H, C, R, *refs):
    i = pl.program_id(0)
    ins = refs[:3 * R + 2]
    outs = refs[3 * R + 2:]
    kw_ref, kb_ref = ins[3 * R], ins[3 * R + 1]
    t_ref = outs[R]
    D = C // H
    for r in range(R):
        n0 = ins[3 * r][...]
        n1 = ins[3 * r + 1][...]
        den = ins[3 * r + 2][...]
        num = jnp.concatenate([n0, n1], axis=1)
        if H > 1:
            num3 = num.reshape(-1, H, D)
            st3 = num3 / (den[:, :, None] + 1e-16)
            st = jnp.maximum(st3.reshape(-1, C), 0.0)
        else:
            st = jnp.maximum(num / (den + 1e-16), 0.0)
        outs[r][...] = st
        tt = jnp.tanh(jnp.dot(st, kw_ref[...], preferred_element_type=jnp.float32)
                      + kb_ref[...])
        tsum = jnp.sum(tt, axis=0, keepdims=True)

        @pl.when(i == 0)
        def _(r=r, tsum=tsum):
            t_ref[r, :] = tsum[0]

        @pl.when(i != 0)
        def _(r=r, tsum=tsum):
            t_ref[r, :] = t_ref[r, :] + tsum[0]


def _norm_t(nums, dens, kw, kb, H, C):
    R = len(nums)
    Dh = C // 2
    in_specs = []
    args = []
    for (n0, n1), den in zip(nums, dens):
        in_specs += [pl.BlockSpec((BM, Dh), lambda i: (i, 0)),
                     pl.BlockSpec((BM, Dh), lambda i: (i, 0)),
                     pl.BlockSpec((BM, H), lambda i: (i, 0))]
        args += [n0, n1, den]
    in_specs += [pl.BlockSpec((C, C), lambda i: (0, 0)),
                 pl.BlockSpec((1, C), lambda i: (0, 0))]
    args += [kw, kb]
    out_specs = [pl.BlockSpec((BM, C), lambda i: (i, 0)) for _ in range(R)]
    out_specs += [pl.BlockSpec((R, C), lambda i: (0, 0))]
    out_shape = [jax.ShapeDtypeStruct((N_NODE, C), jnp.float32) for _ in range(R)]
    out_shape += [jax.ShapeDtypeStruct((R, C), jnp.float32)]
    res = pl.pallas_call(
        functools.partial(_norm_t_body, H, C, R),
        grid=(GRID_M,),
        in_specs=in_specs,
        out_specs=out_specs,
        out_shape=out_shape,
    )(*args)
    return list(res[:R]), res[R]


def _mix2_body(stb0_ref, stb1_ref, tb_ref, q1_ref,
               std_ref, td_ref,
               wb2_ref, bb2_ref, wd2_ref, bd2_ref, lsb2_ref, lsd2_ref,
               xb2_ref, xd2_ref, ab2_ref, ad2_ref, gb2_ref, gd2_ref):
    i = pl.program_id(0)
    q = q1_ref[...]
    tb = tb_ref[...] * (1.0 / N_NODE)
    s0 = jnp.sum(q[0] * tb[0])
    s1 = jnp.sum(q[0] * tb[1])
    m = jnp.maximum(s0, s1)
    e0 = jnp.exp(s0 - m)
    e1 = jnp.exp(s1 - m)
    inv = 1.0 / (e0 + e1)
    hb = stb0_ref[...] * (e0 * inv) + stb1_ref[...] * (e1 * inv)
    hb = jnp.where(hb > 0, hb, jnp.exp(jnp.minimum(hb, 0.0)) - 1.0)
    hd = std_ref[...]
    hd = jnp.where(hd > 0, hd, jnp.exp(jnp.minimum(hd, 0.0)) - 1.0)
    del td_ref
    xb2 = jnp.dot(hb, wb2_ref[...], preferred_element_type=jnp.float32) + bb2_ref[...]
    xd2 = jnp.dot(hd, wd2_ref[...], preferred_element_type=jnp.float32) + bd2_ref[...]
    xb2_ref[...] = xb2
    xd2_ref[...] = xd2
    ab2 = jnp.dot(xb2, lsb2_ref[...], preferred_element_type=jnp.float32)
    ad2 = jnp.dot(xd2, lsd2_ref[...], preferred_element_type=jnp.float32)
    ab2_ref[...] = ab2
    ad2_ref[...] = ad2
    gb2 = jnp.max(ab2, axis=0, keepdims=True)
    gd2 = jnp.max(ad2, axis=0, keepdims=True)

    @pl.when(i == 0)
    def _():
        gb2_ref[...] = gb2
        gd2_ref[...] = gd2

    @pl.when(i != 0)
    def _():
        gb2_ref[...] = jnp.maximum(gb2_ref[...], gb2)
        gd2_ref[...] = jnp.maximum(gd2_ref[...], gd2)


def _mix2(stb, tb, q1, std, td, wb2, bb2, wd2, bd2, lsb2, lsd2):
    return pl.pallas_call(
        _mix2_body,
        grid=(GRID_M,),
        in_specs=[
            pl.BlockSpec((BM, 256), lambda i: (i, 0)),
            pl.BlockSpec((BM, 256), lambda i: (i, 0)),
            pl.BlockSpec((2, 256), lambda i: (0, 0)),
            pl.BlockSpec((1, 256), lambda i: (0, 0)),
            pl.BlockSpec((BM, 256), lambda i: (i, 0)),
            pl.BlockSpec((1, 256), lambda i: (0, 0)),
            pl.BlockSpec((256, 128), lambda i: (0, 0)),
            pl.BlockSpec((1, 128), lambda i: (0, 0)),
            pl.BlockSpec((256, 128), lambda i: (0, 0)),
            pl.BlockSpec((1, 128), lambda i: (0, 0)),
            pl.BlockSpec((128, 3), lambda i: (0, 0)),
            pl.BlockSpec((128, 1), lambda i: (0, 0)),
        ],
        out_specs=[
            pl.BlockSpec((BM, 128), lambda i: (i, 0)),
            pl.BlockSpec((BM, 128), lambda i: (i, 0)),
            pl.BlockSpec((BM, 3), lambda i: (i, 0)),
            pl.BlockSpec((BM, 1), lambda i: (i, 0)),
            pl.BlockSpec((1, 3), lambda i: (0, 0)),
            pl.BlockSpec((1, 1), lambda i: (0, 0)),
        ],
        out_shape=[
            jax.ShapeDtypeStruct((N_NODE, 128), jnp.float32),
            jax.ShapeDtypeStruct((N_NODE, 128), jnp.float32),
            jax.ShapeDtypeStruct((N_NODE, 3), jnp.float32),
            jax.ShapeDtypeStruct((N_NODE, 1), jnp.float32),
            jax.ShapeDtypeStruct((1, 3), jnp.float32),
            jax.ShapeDtypeStruct((1, 1), jnp.float32),
        ],
    )(stb[0], stb[1], tb, q1, std, td, wb2, bb2, wd2, bd2, lsb2, lsd2)


def _final_body(st0_ref, st1_ref, t_ref, q2_ref, cw_ref, cb_ref, out_ref):
    q = q2_ref[...]
    t = t_ref[...] * (1.0 / N_NODE)
    s0 = jnp.sum(q[0] * t[0])
    s1 = jnp.sum(q[0] * t[1])
    m = jnp.maximum(s0, s1)
    e0 = jnp.exp(s0 - m)
    e1 = jnp.exp(s1 - m)
    inv = 1.0 / (e0 + e1)
    h = st0_ref[...] * (e0 * inv) + st1_ref[...] * (e1 * inv)
    out_ref[...] = (jnp.dot(h, cw_ref[...], preferred_element_type=jnp.float32)
                    + cb_ref[...])


def _final(st, t, q2, cw, cb):
    return pl.pallas_call(
        _final_body,
        grid=(GRID_M,),
        in_specs=[
            pl.BlockSpec((BM, 128), lambda i: (i, 0)),
            pl.BlockSpec((BM, 128), lambda i: (i, 0)),
            pl.BlockSpec((2, 128), lambda i: (0, 0)),
            pl.BlockSpec((1, 128), lambda i: (0, 0)),
            pl.BlockSpec((128, 10), lambda i: (0, 0)),
            pl.BlockSpec((1, 10), lambda i: (0, 0)),
        ],
        out_specs=[pl.BlockSpec((BM, 10), lambda i: (i, 0))],
        out_shape=[jax.ShapeDtypeStruct((N_NODE, 10), jnp.float32)],
    )(st[0], st[1], t, q2, cw, cb)[0]


def _edge_stage(xboth, a_src, a_dst, gvec, row, col, n_src, n_dst, H, C):
    Dh = C // 2
    x0 = xboth[:n_src]
    x1 = xboth[n_src:]
    asv = a_src[row]
    adv = a_dst[col]
    s = asv + adv
    lr = jnp.maximum(s, 0.2 * s)
    g = gvec[:H][None, :] + adv
    cb = jnp.maximum(g, 0.2 * g)
    w = jnp.exp(lr - cb)
    den = jax.ops.segment_sum(w, col, num_segments=n_dst)
    D = C // H
    if H > 1:
        w0 = jnp.repeat(w[:, :H // 2], D, axis=1)
        w1 = jnp.repeat(w[:, H // 2:], D, axis=1)
    else:
        w0 = jnp.repeat(w, Dh, axis=1)
        w1 = w0
    num0 = jax.ops.segment_sum(x0[row] * w0, col, num_segments=n_dst)
    num1 = jax.ops.segment_sum(x1[row] * w1, col, num_segments=n_dst)
    return num0, num1, den


def _ls_mat(ls, H, D):
    return (ls[0][:, :, None] * jnp.eye(H, dtype=jnp.float32)[:, None, :]).reshape(H * D, H)


def kernel(x_bug, x_dev, ei_bug_to_dev, ei_dev_to_bug, ei_bug_dup_bug, params):
    p1 = params["han1"]
    p2 = params["han2"]

    lsb = jnp.concatenate([
        _ls_mat(p1["lin_src"]["bug__to__dev"], 8, 32),
        _ls_mat(p1["lin_dst"]["dev__to__bug"], 8, 32),
        _ls_mat(p1["lin_src"]["bug__dup__bug"], 8, 32),
        _ls_mat(p1["lin_dst"]["bug__dup__bug"], 8, 32),
    ], axis=1)
    lsd = jnp.concatenate([
        _ls_mat(p1["lin_dst"]["bug__to__dev"], 8, 32),
        _ls_mat(p1["lin_src"]["dev__to__bug"], 8, 32),
    ], axis=1)
    xb, xd, ab, ad, gb, gd = _proj1(
        x_bug, x_dev,
        p1["proj"]["bug"]["W"], p1["proj"]["bug"]["b"].reshape(1, 256),
        p1["proj"]["dev"]["W"], p1["proj"]["dev"]["b"].reshape(1, 256),
        lsb, lsd)

    xb_both = jnp.concatenate([xb[:, :128], xb[:, 128:]], axis=0)
    xd_both = jnp.concatenate([xd[:, :128], xd[:, 128:]], axis=0)

    def gv8(g):
        return jnp.tile(g.reshape(-1), 2)

    n_b2d = _edge_stage(xb_both, ab[:, 0:8], ad[:, 0:8], gv8(gb[:, 0:8]),
                        ei_bug_to_dev[0], ei_bug_to_dev[1], N_NODE, N_NODE, 8, 256)
    n_d2b = _edge_stage(xd_both, ad[:, 8:16], ab[:, 8:16], gv8(gd[:, 8:16]),
                        ei_dev_to_bug[0], ei_dev_to_bug[1], N_NODE, N_NODE, 8, 256)
    n_dup = _edge_stage(xb_both, ab[:, 16:24], ab[:, 24:32], gv8(gb[:, 16:24]),
                        ei_bug_dup_bug[0], ei_bug_dup_bug[1], N_NODE, N_NODE, 8, 256)

    stb, tb = _norm_t([(n_d2b[0], n_d2b[1]), (n_dup[0], n_dup[1])],
                      [n_d2b[2], n_dup[2]],
                      p1["k_lin"]["W"], p1["k_lin"]["b"].reshape(1, 256), 8, 256)
    std, td = _norm_t([(n_b2d[0], n_b2d[1])], [n_b2d[2]],
                      p1["k_lin"]["W"], p1["k_lin"]["b"].reshape(1, 256), 8, 256)

    lsb2 = jnp.concatenate([
        _ls_mat(p2["lin_src"]["bug__dup__bug"], 1, 128),
        _ls_mat(p2["lin_dst"]["bug__dup__bug"], 1, 128),
        _ls_mat(p2["lin_dst"]["dev__to__bug"], 1, 128),
    ], axis=1)
    lsd2 = _ls_mat(p2["lin_src"]["dev__to__bug"], 1, 128)
    xb2, xd2, ab2, ad2, gb2, gd2 = _mix2(
        stb, tb, p1["q"], std[0], td,
        p2["proj"]["bug"]["W"], p2["proj"]["bug"]["b"].reshape(1, 128),
        p2["proj"]["dev"]["W"], p2["proj"]["dev"]["b"].reshape(1, 128),
        lsb2, lsd2)

    xb2_both = jnp.concatenate([xb2[:, :64], xb2[:, 64:]], axis=0)
    xd2_both = jnp.concatenate([xd2[:, :64], xd2[:, 64:]], axis=0)

    g_d2b2 = jnp.tile(gd2.reshape(1), 16)
    g_dup2 = jnp.tile(gb2[:, 0].reshape(1), 16)

    n2_d2b = _edge_stage(xd2_both, ad2[:, 0:1], ab2[:, 2:3], g_d2b2,
                         ei_dev_to_bug[0], ei_dev_to_bug[1], N_NODE, N_NODE, 1, 128)
    n2_dup = _edge_stage(xb2_both, ab2[:, 0:1], ab2[:, 1:2], g_dup2,
                         ei_bug_dup_bug[0], ei_bug_dup_bug[1], N_NODE, N_NODE, 1, 128)

    st2, t2 = _norm_t([(n2_d2b[0], n2_d2b[1]), (n2_dup[0], n2_dup[1])],
                      [n2_d2b[2], n2_dup[2]],
                      p2["k_lin"]["W"], p2["k_lin"]["b"].reshape(1, 128), 1, 128)

    return _final(st2, t2, p2["q"], params["cls"]["W"],
                  params["cls"]["b"].reshape(1, 10))


# speedup vs baseline: 8.0412x; 1.6075x over previous
import functools

import jax
import jax.numpy as jnp
from jax import lax
from jax.experimental import pallas as pl
from jax.experimental.pallas import tpu as pltpu
from jax.experimental.pallas import tpu_sc as plsc

N_NODE = 10000
E = 160000
BM = 400
GRID_M = N_NODE // BM


def _proj1_body(xb_ref, xd_ref, wb_ref, bb_ref, wd_ref, bd_ref, lsb_ref, lsd_ref,
                xbo_ref, xdo_ref, ab_ref, ad_ref, gb_ref, gd_ref):
    i = pl.program_id(0)
    xb = jnp.dot(xb_ref[...], wb_ref[...], preferred_element_type=jnp.float32) + bb_ref[...]
    xd = jnp.dot(xd_ref[...], wd_ref[...], preferred_element_type=jnp.float32) + bd_ref[...]
    xbo_ref[...] = xb
    xdo_ref[...] = xd
    ab = jnp.dot(xb, lsb_ref[...], preferred_element_type=jnp.float32)
    ad = jnp.dot(xd, lsd_ref[...], preferred_element_type=jnp.float32)
    ab_ref[...] = ab
    ad_ref[...] = ad
    gb = jnp.max(ab, axis=0, keepdims=True)
    gd = jnp.max(ad, axis=0, keepdims=True)

    @pl.when(i == 0)
    def _():
        gb_ref[...] = gb
        gd_ref[...] = gd

    @pl.when(i != 0)
    def _():
        gb_ref[...] = jnp.maximum(gb_ref[...], gb)
        gd_ref[...] = jnp.maximum(gd_ref[...], gd)


def _proj1(x_bug, x_dev, wb, bb, wd, bd, lsb, lsd):
    return pl.pallas_call(
        _proj1_body,
        grid=(GRID_M,),
        in_specs=[
            pl.BlockSpec((BM, 256), lambda i: (i, 0)),
            pl.BlockSpec((BM, 256), lambda i: (i, 0)),
            pl.BlockSpec((256, 256), lambda i: (0, 0)),
            pl.BlockSpec((1, 256), lambda i: (0, 0)),
            pl.BlockSpec((256, 256), lambda i: (0, 0)),
            pl.BlockSpec((1, 256), lambda i: (0, 0)),
            pl.BlockSpec((256, 32), lambda i: (0, 0)),
            pl.BlockSpec((256, 16), lambda i: (0, 0)),
        ],
        out_specs=[
            pl.BlockSpec((BM, 256), lambda i: (i, 0)),
            pl.BlockSpec((BM, 256), lambda i: (i, 0)),
            pl.BlockSpec((BM, 32), lambda i: (i, 0)),
            pl.BlockSpec((BM, 16), lambda i: (i, 0)),
            pl.BlockSpec((1, 32), lambda i: (0, 0)),
            pl.BlockSpec((1, 16), lambda i: (0, 0)),
        ],
        out_shape=[
            jax.ShapeDtypeStruct((N_NODE, 256), jnp.float32),
            jax.ShapeDtypeStruct((N_NODE, 256), jnp.float32),
            jax.ShapeDtypeStruct((N_NODE, 32), jnp.float32),
            jax.ShapeDtypeStruct((N_NODE, 16), jnp.float32),
            jax.ShapeDtypeStruct((1, 32), jnp.float32),
            jax.ShapeDtypeStruct((1, 16), jnp.float32),
        ],
    )(x_bug, x_dev, wb, bb, wd, bd, lsb, lsd)


def _norm_t_body(H, C, R, *refs):
    i = pl.program_id(0)
    ins = refs[:3 * R + 2]
    outs = refs[3 * R + 2:]
    kw_ref, kb_ref = ins[3 * R], ins[3 * R + 1]
    t_ref = outs[R]
    D = C // H
    for r in range(R):
        n0 = ins[3 * r][...]
        n1 = ins[3 * r + 1][...]
        den = ins[3 * r + 2][...]
        num = jnp.concatenate([n0, n1], axis=1)
        if H > 1:
            num3 = num.reshape(-1, H, D)
            st3 = num3 / (den[:, :, None] + 1e-16)
            st = jnp.maximum(st3.reshape(-1, C), 0.0)
        else:
            st = jnp.maximum(num / (den + 1e-16), 0.0)
        outs[r][...] = st
        tt = jnp.tanh(jnp.dot(st, kw_ref[...], preferred_element_type=jnp.float32)
                      + kb_ref[...])
        tsum = jnp.sum(tt, axis=0, keepdims=True)

        @pl.when(i == 0)
        def _(r=r, tsum=tsum):
            t_ref[r, :] = tsum[0]

        @pl.when(i != 0)
        def _(r=r, tsum=tsum):
            t_ref[r, :] = t_ref[r, :] + tsum[0]


def _norm_t(nums, dens, kw, kb, H, C):
    R = len(nums)
    Dh = C // 2
    in_specs = []
    args = []
    for (n0, n1), den in zip(nums, dens):
        in_specs += [pl.BlockSpec((BM, Dh), lambda i: (i, 0)),
                     pl.BlockSpec((BM, Dh), lambda i: (i, 0)),
                     pl.BlockSpec((BM, H), lambda i: (i, 0))]
        args += [n0, n1, den]
    in_specs += [pl.BlockSpec((C, C), lambda i: (0, 0)),
                 pl.BlockSpec((1, C), lambda i: (0, 0))]
    args += [kw, kb]
    out_specs = [pl.BlockSpec((BM, C), lambda i: (i, 0)) for _ in range(R)]
    out_specs += [pl.BlockSpec((R, C), lambda i: (0, 0))]
    out_shape = [jax.ShapeDtypeStruct((N_NODE, C), jnp.float32) for _ in range(R)]
    out_shape += [jax.ShapeDtypeStruct((R, C), jnp.float32)]
    res = pl.pallas_call(
        functools.partial(_norm_t_body, H, C, R),
        grid=(GRID_M,),
        in_specs=in_specs,
        out_specs=out_specs,
        out_shape=out_shape,
    )(*args)
    return list(res[:R]), res[R]


def _mix2_body(stb0_ref, stb1_ref, tb_ref, q1_ref,
               std_ref, td_ref,
               wb2_ref, bb2_ref, wd2_ref, bd2_ref, lsb2_ref, lsd2_ref,
               xb2_ref, xd2_ref, ab2_ref, ad2_ref, gb2_ref, gd2_ref):
    i = pl.program_id(0)
    q = q1_ref[...]
    tb = tb_ref[...] * (1.0 / N_NODE)
    s0 = jnp.sum(q[0] * tb[0])
    s1 = jnp.sum(q[0] * tb[1])
    m = jnp.maximum(s0, s1)
    e0 = jnp.exp(s0 - m)
    e1 = jnp.exp(s1 - m)
    inv = 1.0 / (e0 + e1)
    hb = stb0_ref[...] * (e0 * inv) + stb1_ref[...] * (e1 * inv)
    hb = jnp.where(hb > 0, hb, jnp.exp(jnp.minimum(hb, 0.0)) - 1.0)
    hd = std_ref[...]
    hd = jnp.where(hd > 0, hd, jnp.exp(jnp.minimum(hd, 0.0)) - 1.0)
    del td_ref
    xb2 = jnp.dot(hb, wb2_ref[...], preferred_element_type=jnp.float32) + bb2_ref[...]
    xd2 = jnp.dot(hd, wd2_ref[...], preferred_element_type=jnp.float32) + bd2_ref[...]
    xb2_ref[...] = xb2
    xd2_ref[...] = xd2
    ab2 = jnp.dot(xb2, lsb2_ref[...], preferred_element_type=jnp.float32)
    ad2 = jnp.dot(xd2, lsd2_ref[...], preferred_element_type=jnp.float32)
    ab2_ref[...] = ab2
    ad2_ref[...] = ad2
    gb2 = jnp.max(ab2, axis=0, keepdims=True)
    gd2 = jnp.max(ad2, axis=0, keepdims=True)

    @pl.when(i == 0)
    def _():
        gb2_ref[...] = gb2
        gd2_ref[...] = gd2

    @pl.when(i != 0)
    def _():
        gb2_ref[...] = jnp.maximum(gb2_ref[...], gb2)
        gd2_ref[...] = jnp.maximum(gd2_ref[...], gd2)


def _mix2(stb, tb, q1, std, td, wb2, bb2, wd2, bd2, lsb2, lsd2):
    return pl.pallas_call(
        _mix2_body,
        grid=(GRID_M,),
        in_specs=[
            pl.BlockSpec((BM, 256), lambda i: (i, 0)),
            pl.BlockSpec((BM, 256), lambda i: (i, 0)),
            pl.BlockSpec((2, 256), lambda i: (0, 0)),
            pl.BlockSpec((1, 256), lambda i: (0, 0)),
            pl.BlockSpec((BM, 256), lambda i: (i, 0)),
            pl.BlockSpec((1, 256), lambda i: (0, 0)),
            pl.BlockSpec((256, 128), lambda i: (0, 0)),
            pl.BlockSpec((1, 128), lambda i: (0, 0)),
            pl.BlockSpec((256, 128), lambda i: (0, 0)),
            pl.BlockSpec((1, 128), lambda i: (0, 0)),
            pl.BlockSpec((128, 3), lambda i: (0, 0)),
            pl.BlockSpec((128, 1), lambda i: (0, 0)),
        ],
        out_specs=[
            pl.BlockSpec((BM, 128), lambda i: (i, 0)),
            pl.BlockSpec((BM, 128), lambda i: (i, 0)),
            pl.BlockSpec((BM, 3), lambda i: (i, 0)),
            pl.BlockSpec((BM, 1), lambda i: (i, 0)),
            pl.BlockSpec((1, 3), lambda i: (0, 0)),
            pl.BlockSpec((1, 1), lambda i: (0, 0)),
        ],
        out_shape=[
            jax.ShapeDtypeStruct((N_NODE, 128), jnp.float32),
            jax.ShapeDtypeStruct((N_NODE, 128), jnp.float32),
            jax.ShapeDtypeStruct((N_NODE, 3), jnp.float32),
            jax.ShapeDtypeStruct((N_NODE, 1), jnp.float32),
            jax.ShapeDtypeStruct((1, 3), jnp.float32),
            jax.ShapeDtypeStruct((1, 1), jnp.float32),
        ],
    )(stb[0], stb[1], tb, q1, std, td, wb2, bb2, wd2, bd2, lsb2, lsd2)


def _final_body(st0_ref, st1_ref, t_ref, q2_ref, cw_ref, cb_ref, out_ref):
    q = q2_ref[...]
    t = t_ref[...] * (1.0 / N_NODE)
    s0 = jnp.sum(q[0] * t[0])
    s1 = jnp.sum(q[0] * t[1])
    m = jnp.maximum(s0, s1)
    e0 = jnp.exp(s0 - m)
    e1 = jnp.exp(s1 - m)
    inv = 1.0 / (e0 + e1)
    h = st0_ref[...] * (e0 * inv) + st1_ref[...] * (e1 * inv)
    out_ref[...] = (jnp.dot(h, cw_ref[...], preferred_element_type=jnp.float32)
                    + cb_ref[...])


def _final(st, t, q2, cw, cb):
    return pl.pallas_call(
        _final_body,
        grid=(GRID_M,),
        in_specs=[
            pl.BlockSpec((BM, 128), lambda i: (i, 0)),
            pl.BlockSpec((BM, 128), lambda i: (i, 0)),
            pl.BlockSpec((2, 128), lambda i: (0, 0)),
            pl.BlockSpec((1, 128), lambda i: (0, 0)),
            pl.BlockSpec((128, 10), lambda i: (0, 0)),
            pl.BlockSpec((1, 10), lambda i: (0, 0)),
        ],
        out_specs=[pl.BlockSpec((BM, 10), lambda i: (i, 0))],
        out_shape=[jax.ShapeDtypeStruct((N_NODE, 10), jnp.float32)],
    )(st[0], st[1], t, q2, cw, cb)[0]


_K = 80
_SUB = 80
_NSUB = _K // _SUB


def _take16(v, idx):
    return lax.gather(
        v, idx[:, None],
        lax.GatherDimensionNumbers(offset_dims=(), collapsed_slice_dims=(0,),
                                   start_index_map=(0,)),
        (1,), mode=lax.GatherScatterMode.PROMISE_IN_BOUNDS)


def _edge_body(n_src, np_dst, H, C, row_hbm, col_hbm, x_hbm, as_hbm, ad_hbm,
               gv_hbm, za_hbm, num_out, den_out,
               rowf, colf, colv2, colg2, rowv2, fidxr, fidxc, asg, adg, wflat,
               wv2, wexp, xrows, gvv, acc, sem):
    Dh = C // 2
    HALF = np_dst // 2
    c = lax.axis_index("c")
    s = lax.axis_index("s")
    EP = E // 16
    nchunk = EP // _K
    RPZ = (HALF + 128) // 16
    RPF = HALF // 16

    pltpu.sync_copy(gv_hbm, gvv)
    pltpu.sync_copy(za_hbm.at[pl.ds(0, _K)], wexp)
    g = gvv[...]
    cn = c * n_src if H == 8 else 0
    il = lax.iota(jnp.int32, 16)
    zl = il & 0
    lo8 = il < 8
    low3 = il & 7
    pair = il >> 3

    if H == 8:
        passes = [("num", 0), ("num", 1), ("den", 0), ("den", 1)]
    else:
        passes = [("num", None), ("den", None)]

    for kind, ph in passes:
        off = ph * HALF if ph is not None else c * HALF

        def iloop(j, _):
            o = s * RPZ + j * 80
            pltpu.sync_copy(za_hbm.at[pl.ds(o, 80)], acc.at[pl.ds(o, 80)])
            return 0

        lax.fori_loop(0, RPZ // 80, iloop, 0)
        o8 = s * RPZ + (RPZ // 80) * 80
        pltpu.sync_copy(za_hbm.at[pl.ds(o8, RPZ % 80)], acc.at[pl.ds(o8, RPZ % 80)])
        plsc.subcore_barrier()

        def chunk(i, carry, kind=kind):
            base = s * EP + i * _K
            pltpu.sync_copy(row_hbm.at[pl.ds(base, _K)], rowf)
            pltpu.sync_copy(col_hbm.at[pl.ds(base, _K)], colf)
            for b in range(_K // 16):
                cv = colf[pl.ds(b * 16, 16)]
                sh = cv - off
                ok = (sh >= 0) & (sh < HALF)
                colv2[0, pl.ds(b * 16, 16)] = jnp.where(ok, sh, HALF)
                colg2[0, pl.ds(b * 16, 16)] = cv
                rowv2[0, pl.ds(b * 16, 16)] = rowf[pl.ds(b * 16, 16)] + cn

            cps = []
            if H == 8:
                def eloop(jb, _):
                    rv = rowf[pl.ds(jb * 16, 16)]
                    cv = colf[pl.ds(jb * 16, 16)]
                    for sub in range(8):
                        idxr = _take16(rv, pair + 2 * sub)
                        idxc = _take16(cv, pair + 2 * sub)
                        fidxr[jb, pl.ds(sub * 16, 16)] = idxr * 8 + low3
                        fidxc[jb, pl.ds(sub * 16, 16)] = idxc * 8 + low3
                    return 0

                lax.fori_loop(0, _K // 16, eloop, 0)
                for t in range(_K * H // 128):
                    sl = pl.ds(t * 128, 128)
                    cps.append(pltpu.async_copy(as_hbm.at[fidxr.at[t]], asg.at[sl], sem))
                    cps.append(pltpu.async_copy(ad_hbm.at[fidxc.at[t]], adg.at[sl], sem))
            else:
                cps.append(pltpu.async_copy(as_hbm.at[rowv2.at[0]], asg.at[pl.ds(0, _K)], sem))
                cps.append(pltpu.async_copy(ad_hbm.at[colg2.at[0]], adg.at[pl.ds(0, _K)], sem))
            if kind == "num":
                cps.append(pltpu.async_copy(x_hbm.at[rowv2.at[0]], xrows.at[pl.ds(0, _K)], sem))
            for cp in cps:
                cp.wait()

            wtgt = wv2 if kind == "num" else wexp
            if H == 8:

                def wloop(j, _, wtgt=wtgt):
                    av = asg[pl.ds(j * 16, 16)]
                    bv = adg[pl.ds(j * 16, 16)]
                    sv = av + bv
                    lr = jnp.maximum(sv, 0.2 * sv)
                    tv = g + bv
                    cb = jnp.maximum(tv, 0.2 * tv)
                    w = jnp.exp(lr - cb)
                    we = _take16(w, low3)
                    wo = _take16(w, low3 + 8)
                    wtgt[2 * j, pl.ds(0, 16)] = jnp.where(lo8, we, 0.0)
                    wtgt[2 * j + 1, pl.ds(0, 16)] = jnp.where(lo8, wo, 0.0)
                    return 0

                lax.fori_loop(0, _K * H // 16, wloop, 0)
            else:

                def wloop(j, _):
                    av = asg[pl.ds(j * 16, 16)]
                    bv = adg[pl.ds(j * 16, 16)]
                    sv = av + bv
                    lr = jnp.maximum(sv, 0.2 * sv)
                    tv = g + bv
                    cb = jnp.maximum(tv, 0.2 * tv)
                    wflat[pl.ds(j * 16, 16)] = jnp.exp(lr - cb)
                    return 0

                lax.fori_loop(0, _K // 16, wloop, 0)

            if kind == "num":
                if H == 8:

                    def mloop(k, _):
                        wrow = wv2[k, pl.ds(0, 16)]
                        for jp in range(Dh // 32):
                            ws = _take16(wrow, zl + (c * (Dh // 32) + jp))
                            xrows[k, pl.ds(jp * 32, 16)] = (
                                xrows[k, pl.ds(jp * 32, 16)] * ws)
                            xrows[k, pl.ds(jp * 32 + 16, 16)] = (
                                xrows[k, pl.ds(jp * 32 + 16, 16)] * ws)
                        return 0
                else:

                    def mloop(k, _):
                        st = pl.multiple_of((k >> 4) * 16, 16)
                        v = wflat[pl.ds(st, 16)]
                        ws = _take16(v, zl + (k & 15))
                        for jp in range(C // 16):
                            xrows[k, pl.ds(jp * 16, 16)] = (
                                xrows[k, pl.ds(jp * 16, 16)] * ws)
                        return 0

                lax.fori_loop(0, _K, mloop, 0)
                pltpu.sync_copy(xrows.at[pl.ds(0, _K)], acc.at[colv2.at[0]], add=True)
            else:
                if H == 1:
                    def dloop(k, _):
                        st = pl.multiple_of((k >> 4) * 16, 16)
                        v = wflat[pl.ds(st, 16)]
                        ws = _take16(v, zl + (k & 15))
                        wexp[k, pl.ds(0, 16)] = jnp.where(lo8, ws, 0.0)
                        return 0

                    lax.fori_loop(0, _K, dloop, 0)
                pltpu.sync_copy(wexp.at[pl.ds(0, _K)], acc.at[colv2.at[0]], add=True)
            return carry

        lax.fori_loop(0, nchunk, chunk, 0)
        plsc.subcore_barrier()

        def floop(j, _, kind=kind):
            o = s * RPF + j * 80
            if kind == "num":
                if H == 8:
                    pltpu.sync_copy(acc.at[pl.ds(o, 80)],
                                    num_out.at[c, pl.ds(off + o, 80)])
                else:
                    pltpu.sync_copy(acc.at[pl.ds(o, 80)],
                                    num_out.at[pl.ds(off + o, 80)])
            else:
                if H == 8:
                    @pl.when(c == 0)
                    def _():
                        pltpu.sync_copy(acc.at[pl.ds(o, 80)],
                                        den_out.at[pl.ds(off + o, 80)])
                else:
                    pltpu.sync_copy(acc.at[pl.ds(o, 80)],
                                    den_out.at[pl.ds(off + o, 80)])
            return 0

        lax.fori_loop(0, RPF // 80, floop, 0)
        plsc.subcore_barrier()


@functools.lru_cache(maxsize=None)
def _edge_kernel(n_src, np_dst, H, C):
    Dh = C // 2
    HALF = np_dst // 2
    mesh = plsc.VectorSubcoreMesh(core_axis_name="c", subcore_axis_name="s")
    if H == 8:
        out_type = [
            jax.ShapeDtypeStruct((2, np_dst, Dh), jnp.float32),
            jax.ShapeDtypeStruct((np_dst, 128), jnp.float32),
        ]
        xw = Dh
    else:
        out_type = [
            jax.ShapeDtypeStruct((np_dst, C), jnp.float32),
            jax.ShapeDtypeStruct((np_dst, 128), jnp.float32),
        ]
        xw = C
    return pl.kernel(
        functools.partial(_edge_body, n_src, np_dst, H, C),
        out_type=out_type,
        mesh=mesh,
        scratch_types=[
            pltpu.VMEM((_K,), jnp.int32),
            pltpu.VMEM((_K,), jnp.int32),
            pltpu.VMEM((1, _K), jnp.int32),
            pltpu.VMEM((1, _K), jnp.int32),
            pltpu.VMEM((1, _K), jnp.int32),
            pltpu.VMEM((_K * H // 128 if H == 8 else 1, 128), jnp.int32),
            pltpu.VMEM((_K * H // 128 if H == 8 else 1, 128), jnp.int32),
            pltpu.VMEM((_K * H,), jnp.float32),
            pltpu.VMEM((_K * H,), jnp.float32),
            pltpu.VMEM((_K,), jnp.float32),
            pltpu.VMEM((_K, 16), jnp.float32),
            pltpu.VMEM((_K, 128), jnp.float32),
            pltpu.VMEM((_K, xw), jnp.float32),
            pltpu.VMEM((16,), jnp.float32),
            pltpu.VMEM_SHARED((HALF + 128, 128), jnp.float32),
            pltpu.SemaphoreType.DMA,
        ],
    )


def _edge_stage(xtab, a_src, a_dst, gvec, row, col, n_src, n_dst, H, C):
    Dh = C // 2
    np_dst = -(-n_dst // 256) * 256
    HALF = np_dst // 2
    k = _edge_kernel(n_src, np_dst, H, C)
    za = jnp.zeros((HALF + 128, 128), jnp.float32)
    num, den = k(row, col, xtab, a_src.reshape(-1), a_dst.reshape(-1),
                 gvec, za)
    if H == 8:
        return num[0, :n_dst], num[1, :n_dst], den[:n_dst, :H]
    return num[:n_dst, :Dh], num[:n_dst, Dh:], den[:n_dst, :H]


def _ls_mat(ls, H, D):
    return (ls[0][:, :, None] * jnp.eye(H, dtype=jnp.float32)[:, None, :]).reshape(H * D, H)


def kernel(x_bug, x_dev, ei_bug_to_dev, ei_dev_to_bug, ei_bug_dup_bug, params):
    p1 = params["han1"]
    p2 = params["han2"]

    lsb = jnp.concatenate([
        _ls_mat(p1["lin_src"]["bug__to__dev"], 8, 32),
        _ls_mat(p1["lin_dst"]["dev__to__bug"], 8, 32),
        _ls_mat(p1["lin_src"]["bug__dup__bug"], 8, 32),
        _ls_mat(p1["lin_dst"]["bug__dup__bug"], 8, 32),
    ], axis=1)
    lsd = jnp.concatenate([
        _ls_mat(p1["lin_dst"]["bug__to__dev"], 8, 32),
        _ls_mat(p1["lin_src"]["dev__to__bug"], 8, 32),
    ], axis=1)
    xb, xd, ab, ad, gb, gd = _proj1(
        x_bug, x_dev,
        p1["proj"]["bug"]["W"], p1["proj"]["bug"]["b"].reshape(1, 256),
        p1["proj"]["dev"]["W"], p1["proj"]["dev"]["b"].reshape(1, 256),
        lsb, lsd)

    xb_both = jnp.concatenate([xb[:, :128], xb[:, 128:]], axis=0)
    xd_both = jnp.concatenate([xd[:, :128], xd[:, 128:]], axis=0)

    def gv8(g):
        return jnp.tile(g.reshape(-1), 2)

    n_b2d = _edge_stage(xb_both, ab[:, 0:8], ad[:, 0:8], gv8(gb[:, 0:8]),
                        ei_bug_to_dev[0], ei_bug_to_dev[1], N_NODE, N_NODE, 8, 256)
    n_d2b = _edge_stage(xd_both, ad[:, 8:16], ab[:, 8:16], gv8(gd[:, 8:16]),
                        ei_dev_to_bug[0], ei_dev_to_bug[1], N_NODE, N_NODE, 8, 256)
    n_dup = _edge_stage(xb_both, ab[:, 16:24], ab[:, 24:32], gv8(gb[:, 16:24]),
                        ei_bug_dup_bug[0], ei_bug_dup_bug[1], N_NODE, N_NODE, 8, 256)

    stb, tb = _norm_t([(n_d2b[0], n_d2b[1]), (n_dup[0], n_dup[1])],
                      [n_d2b[2], n_dup[2]],
                      p1["k_lin"]["W"], p1["k_lin"]["b"].reshape(1, 256), 8, 256)
    std, td = _norm_t([(n_b2d[0], n_b2d[1])], [n_b2d[2]],
                      p1["k_lin"]["W"], p1["k_lin"]["b"].reshape(1, 256), 8, 256)

    lsb2 = jnp.concatenate([
        _ls_mat(p2["lin_src"]["bug__dup__bug"], 1, 128),
        _ls_mat(p2["lin_dst"]["bug__dup__bug"], 1, 128),
        _ls_mat(p2["lin_dst"]["dev__to__bug"], 1, 128),
    ], axis=1)
    lsd2 = _ls_mat(p2["lin_src"]["dev__to__bug"], 1, 128)
    xb2, xd2, ab2, ad2, gb2, gd2 = _mix2(
        stb, tb, p1["q"], std[0], td,
        p2["proj"]["bug"]["W"], p2["proj"]["bug"]["b"].reshape(1, 128),
        p2["proj"]["dev"]["W"], p2["proj"]["dev"]["b"].reshape(1, 128),
        lsb2, lsd2)

    g_d2b2 = jnp.tile(gd2.reshape(1), 16)
    g_dup2 = jnp.tile(gb2[:, 0].reshape(1), 16)

    n2_d2b = _edge_stage(xd2, ad2[:, 0:1], ab2[:, 2:3], g_d2b2,
                         ei_dev_to_bug[0], ei_dev_to_bug[1], N_NODE, N_NODE, 1, 128)
    n2_dup = _edge_stage(xb2, ab2[:, 0:1], ab2[:, 1:2], g_dup2,
                         ei_bug_dup_bug[0], ei_bug_dup_bug[1], N_NODE, N_NODE, 1, 128)

    st2, t2 = _norm_t([(n2_d2b[0], n2_d2b[1]), (n2_dup[0], n2_dup[1])],
                      [n2_d2b[2], n2_dup[2]],
                      p2["k_lin"]["W"], p2["k_lin"]["b"].reshape(1, 128), 1, 128)

    return _final(st2, t2, p2["q"], params["cls"]["W"],
                  params["cls"]["b"].reshape(1, 10))


# speedup vs baseline: 10.8907x; 1.3544x over previous
import functools

import jax
import jax.numpy as jnp
from jax import lax
from jax.experimental import pallas as pl
from jax.experimental.pallas import tpu as pltpu
from jax.experimental.pallas import tpu_sc as plsc

N_NODE = 10000
E = 160000
BM = 400
GRID_M = N_NODE // BM


def _proj1_body(xb_ref, xd_ref, wb_ref, bb_ref, wd_ref, bd_ref, lsb_ref, lsd_ref,
                xbo_ref, xdo_ref, ab_ref, ad_ref, gb_ref, gd_ref):
    i = pl.program_id(0)
    xb = jnp.dot(xb_ref[...], wb_ref[...], preferred_element_type=jnp.float32) + bb_ref[...]
    xd = jnp.dot(xd_ref[...], wd_ref[...], preferred_element_type=jnp.float32) + bd_ref[...]
    xbo_ref[...] = xb
    xdo_ref[...] = xd
    ab = jnp.dot(xb, lsb_ref[...], preferred_element_type=jnp.float32)
    ad = jnp.dot(xd, lsd_ref[...], preferred_element_type=jnp.float32)
    ab_ref[...] = ab
    ad_ref[...] = ad
    gb = jnp.max(ab, axis=0, keepdims=True)
    gd = jnp.max(ad, axis=0, keepdims=True)

    @pl.when(i == 0)
    def _():
        gb_ref[...] = gb
        gd_ref[...] = gd

    @pl.when(i != 0)
    def _():
        gb_ref[...] = jnp.maximum(gb_ref[...], gb)
        gd_ref[...] = jnp.maximum(gd_ref[...], gd)


def _proj1(x_bug, x_dev, wb, bb, wd, bd, lsb, lsd):
    return pl.pallas_call(
        _proj1_body,
        grid=(GRID_M,),
        in_specs=[
            pl.BlockSpec((BM, 256), lambda i: (i, 0)),
            pl.BlockSpec((BM, 256), lambda i: (i, 0)),
            pl.BlockSpec((256, 256), lambda i: (0, 0)),
            pl.BlockSpec((1, 256), lambda i: (0, 0)),
            pl.BlockSpec((256, 256), lambda i: (0, 0)),
            pl.BlockSpec((1, 256), lambda i: (0, 0)),
            pl.BlockSpec((256, 32), lambda i: (0, 0)),
            pl.BlockSpec((256, 16), lambda i: (0, 0)),
        ],
        out_specs=[
            pl.BlockSpec((BM, 256), lambda i: (i, 0)),
            pl.BlockSpec((BM, 256), lambda i: (i, 0)),
            pl.BlockSpec((BM, 32), lambda i: (i, 0)),
            pl.BlockSpec((BM, 16), lambda i: (i, 0)),
            pl.BlockSpec((1, 32), lambda i: (0, 0)),
            pl.BlockSpec((1, 16), lambda i: (0, 0)),
        ],
        out_shape=[
            jax.ShapeDtypeStruct((N_NODE, 256), jnp.float32),
            jax.ShapeDtypeStruct((N_NODE, 256), jnp.float32),
            jax.ShapeDtypeStruct((N_NODE, 32), jnp.float32),
            jax.ShapeDtypeStruct((N_NODE, 16), jnp.float32),
            jax.ShapeDtypeStruct((1, 32), jnp.float32),
            jax.ShapeDtypeStruct((1, 16), jnp.float32),
        ],
    )(x_bug, x_dev, wb, bb, wd, bd, lsb, lsd)


def _norm_t_body(H, C, R, *refs):
    i = pl.program_id(0)
    ins = refs[:3 * R + 2]
    outs = refs[3 * R + 2:]
    kw_ref, kb_ref = ins[3 * R], ins[3 * R + 1]
    t_ref = outs[R]
    D = C // H
    for r in range(R):
        n0 = ins[3 * r][...]
        n1 = ins[3 * r + 1][...]
        den = ins[3 * r + 2][...]
        num = jnp.concatenate([n0, n1], axis=1)
        if H > 1:
            num3 = num.reshape(-1, H, D)
            st3 = num3 / (den[:, :, None] + 1e-16)
            st = jnp.maximum(st3.reshape(-1, C), 0.0)
        else:
            st = jnp.maximum(num / (den + 1e-16), 0.0)
        outs[r][...] = st
        tt = jnp.tanh(jnp.dot(st, kw_ref[...], preferred_element_type=jnp.float32)
                      + kb_ref[...])
        tsum = jnp.sum(tt, axis=0, keepdims=True)

        @pl.when(i == 0)
        def _(r=r, tsum=tsum):
            t_ref[r, :] = tsum[0]

        @pl.when(i != 0)
        def _(r=r, tsum=tsum):
            t_ref[r, :] = t_ref[r, :] + tsum[0]


def _norm_t(nums, dens, kw, kb, H, C):
    R = len(nums)
    Dh = C // 2
    in_specs = []
    args = []
    for (n0, n1), den in zip(nums, dens):
        in_specs += [pl.BlockSpec((BM, Dh), lambda i: (i, 0)),
                     pl.BlockSpec((BM, Dh), lambda i: (i, 0)),
                     pl.BlockSpec((BM, H), lambda i: (i, 0))]
        args += [n0, n1, den]
    in_specs += [pl.BlockSpec((C, C), lambda i: (0, 0)),
                 pl.BlockSpec((1, C), lambda i: (0, 0))]
    args += [kw, kb]
    out_specs = [pl.BlockSpec((BM, C), lambda i: (i, 0)) for _ in range(R)]
    out_specs += [pl.BlockSpec((R, C), lambda i: (0, 0))]
    out_shape = [jax.ShapeDtypeStruct((N_NODE, C), jnp.float32) for _ in range(R)]
    out_shape += [jax.ShapeDtypeStruct((R, C), jnp.float32)]
    res = pl.pallas_call(
        functools.partial(_norm_t_body, H, C, R),
        grid=(GRID_M,),
        in_specs=in_specs,
        out_specs=out_specs,
        out_shape=out_shape,
    )(*args)
    return list(res[:R]), res[R]


def _mix2_body(stb0_ref, stb1_ref, tb_ref, q1_ref,
               std_ref, td_ref,
               wb2_ref, bb2_ref, wd2_ref, bd2_ref, lsb2_ref, lsd2_ref,
               xb2_ref, xd2_ref, ab2_ref, ad2_ref, gb2_ref, gd2_ref):
    i = pl.program_id(0)
    q = q1_ref[...]
    tb = tb_ref[...] * (1.0 / N_NODE)
    s0 = jnp.sum(q[0] * tb[0])
    s1 = jnp.sum(q[0] * tb[1])
    m = jnp.maximum(s0, s1)
    e0 = jnp.exp(s0 - m)
    e1 = jnp.exp(s1 - m)
    inv = 1.0 / (e0 + e1)
    hb = stb0_ref[...] * (e0 * inv) + stb1_ref[...] * (e1 * inv)
    hb = jnp.where(hb > 0, hb, jnp.exp(jnp.minimum(hb, 0.0)) - 1.0)
    hd = std_ref[...]
    hd = jnp.where(hd > 0, hd, jnp.exp(jnp.minimum(hd, 0.0)) - 1.0)
    del td_ref
    xb2 = jnp.dot(hb, wb2_ref[...], preferred_element_type=jnp.float32) + bb2_ref[...]
    xd2 = jnp.dot(hd, wd2_ref[...], preferred_element_type=jnp.float32) + bd2_ref[...]
    xb2_ref[...] = xb2
    xd2_ref[...] = xd2
    ab2 = jnp.dot(xb2, lsb2_ref[...], preferred_element_type=jnp.float32)
    ad2 = jnp.dot(xd2, lsd2_ref[...], preferred_element_type=jnp.float32)
    ab2_ref[...] = ab2
    ad2_ref[...] = ad2
    gb2 = jnp.max(ab2, axis=0, keepdims=True)
    gd2 = jnp.max(ad2, axis=0, keepdims=True)

    @pl.when(i == 0)
    def _():
        gb2_ref[...] = gb2
        gd2_ref[...] = gd2

    @pl.when(i != 0)
    def _():
        gb2_ref[...] = jnp.maximum(gb2_ref[...], gb2)
        gd2_ref[...] = jnp.maximum(gd2_ref[...], gd2)


def _mix2(stb, tb, q1, std, td, wb2, bb2, wd2, bd2, lsb2, lsd2):
    return pl.pallas_call(
        _mix2_body,
        grid=(GRID_M,),
        in_specs=[
            pl.BlockSpec((BM, 256), lambda i: (i, 0)),
            pl.BlockSpec((BM, 256), lambda i: (i, 0)),
            pl.BlockSpec((2, 256), lambda i: (0, 0)),
            pl.BlockSpec((1, 256), lambda i: (0, 0)),
            pl.BlockSpec((BM, 256), lambda i: (i, 0)),
            pl.BlockSpec((1, 256), lambda i: (0, 0)),
            pl.BlockSpec((256, 128), lambda i: (0, 0)),
            pl.BlockSpec((1, 128), lambda i: (0, 0)),
            pl.BlockSpec((256, 128), lambda i: (0, 0)),
            pl.BlockSpec((1, 128), lambda i: (0, 0)),
            pl.BlockSpec((128, 3), lambda i: (0, 0)),
            pl.BlockSpec((128, 1), lambda i: (0, 0)),
        ],
        out_specs=[
            pl.BlockSpec((BM, 128), lambda i: (i, 0)),
            pl.BlockSpec((BM, 128), lambda i: (i, 0)),
            pl.BlockSpec((BM, 3), lambda i: (i, 0)),
            pl.BlockSpec((BM, 1), lambda i: (i, 0)),
            pl.BlockSpec((1, 3), lambda i: (0, 0)),
            pl.BlockSpec((1, 1), lambda i: (0, 0)),
        ],
        out_shape=[
            jax.ShapeDtypeStruct((N_NODE, 128), jnp.float32),
            jax.ShapeDtypeStruct((N_NODE, 128), jnp.float32),
            jax.ShapeDtypeStruct((N_NODE, 3), jnp.float32),
            jax.ShapeDtypeStruct((N_NODE, 1), jnp.float32),
            jax.ShapeDtypeStruct((1, 3), jnp.float32),
            jax.ShapeDtypeStruct((1, 1), jnp.float32),
        ],
    )(stb[0], stb[1], tb, q1, std, td, wb2, bb2, wd2, bd2, lsb2, lsd2)


def _final_body(st0_ref, st1_ref, t_ref, q2_ref, cw_ref, cb_ref, out_ref):
    q = q2_ref[...]
    t = t_ref[...] * (1.0 / N_NODE)
    s0 = jnp.sum(q[0] * t[0])
    s1 = jnp.sum(q[0] * t[1])
    m = jnp.maximum(s0, s1)
    e0 = jnp.exp(s0 - m)
    e1 = jnp.exp(s1 - m)
    inv = 1.0 / (e0 + e1)
    h = st0_ref[...] * (e0 * inv) + st1_ref[...] * (e1 * inv)
    out_ref[...] = (jnp.dot(h, cw_ref[...], preferred_element_type=jnp.float32)
                    + cb_ref[...])


def _final(st, t, q2, cw, cb):
    return pl.pallas_call(
        _final_body,
        grid=(GRID_M,),
        in_specs=[
            pl.BlockSpec((BM, 128), lambda i: (i, 0)),
            pl.BlockSpec((BM, 128), lambda i: (i, 0)),
            pl.BlockSpec((2, 128), lambda i: (0, 0)),
            pl.BlockSpec((1, 128), lambda i: (0, 0)),
            pl.BlockSpec((128, 10), lambda i: (0, 0)),
            pl.BlockSpec((1, 10), lambda i: (0, 0)),
        ],
        out_specs=[pl.BlockSpec((BM, 10), lambda i: (i, 0))],
        out_shape=[jax.ShapeDtypeStruct((N_NODE, 10), jnp.float32)],
    )(st[0], st[1], t, q2, cw, cb)[0]


_K = 80
_SUB = 80
_NSUB = _K // _SUB


def _take16(v, idx):
    return lax.gather(
        v, idx[:, None],
        lax.GatherDimensionNumbers(offset_dims=(), collapsed_slice_dims=(0,),
                                   start_index_map=(0,)),
        (1,), mode=lax.GatherScatterMode.PROMISE_IN_BOUNDS)


def _edge_body(n_src, np_dst, H, C, row_hbm, col_hbm, x_hbm, as_hbm, ad_hbm,
               gv_hbm, za_hbm, num_out, den_out, w_hbm,
               rowf, colf, colv2, colg2, rowv2, fidxr, fidxc, asg, adg, wbuf,
               wv2, wexp, xrows, gvv, acc, sem):
    Dh = C // 2
    HALF = np_dst // 2
    c = lax.axis_index("c")
    s = lax.axis_index("s")
    EP = E // 16
    nchunk = EP // _K
    RPZ = (HALF + 128) // 16
    RPF = HALF // 16

    pltpu.sync_copy(gv_hbm, gvv)
    pltpu.sync_copy(za_hbm.at[pl.ds(0, _K)], wexp)
    g = gvv[...]
    cn = c * n_src if H == 8 else 0
    il = lax.iota(jnp.int32, 16)
    zl = il & 0
    lo8 = il < 8
    low3 = il & 7
    pair = il >> 3

    if H == 8:
        passes = [("num", 0), ("num", 1), ("den", 0), ("den", 1)]
    else:
        passes = [("num", None), ("den", None)]

    EPC = _K * H

    for pi, (kind, ph) in enumerate(passes):
        first = pi == 0
        off = ph * HALF if ph is not None else c * HALF

        def iloop(j, _):
            o = s * RPZ + j * 80
            pltpu.sync_copy(za_hbm.at[pl.ds(o, 80)], acc.at[pl.ds(o, 80)])
            return 0

        lax.fori_loop(0, RPZ // 80, iloop, 0)
        o8 = s * RPZ + (RPZ // 80) * 80
        pltpu.sync_copy(za_hbm.at[pl.ds(o8, RPZ % 80)], acc.at[pl.ds(o8, RPZ % 80)])
        plsc.subcore_barrier()

        def chunk(i, carry, kind=kind):
            base = s * EP + i * _K
            pltpu.sync_copy(row_hbm.at[pl.ds(base, _K)], rowf)
            pltpu.sync_copy(col_hbm.at[pl.ds(base, _K)], colf)
            for b in range(_K // 16):
                cv = colf[pl.ds(b * 16, 16)]
                sh = cv - off
                ok = (sh >= 0) & (sh < HALF)
                a0, b0 = b // (_SUB // 16), (b % (_SUB // 16)) * 16
                colv2[a0, pl.ds(b0, 16)] = jnp.where(ok, sh, HALF)
                colg2[a0, pl.ds(b0, 16)] = cv
                rowv2[a0, pl.ds(b0, 16)] = rowf[pl.ds(b * 16, 16)] + cn

            cps = []
            if H == 8 and first:
                def eloop(jb, _):
                    rv = rowf[pl.ds(jb * 16, 16)]
                    cv = colf[pl.ds(jb * 16, 16)]
                    for sub in range(8):
                        idxr = _take16(rv, pair + 2 * sub)
                        idxc = _take16(cv, pair + 2 * sub)
                        fidxr[jb, pl.ds(sub * 16, 16)] = idxr * 8 + low3
                        fidxc[jb, pl.ds(sub * 16, 16)] = idxc * 8 + low3
                    return 0

                lax.fori_loop(0, _K // 16, eloop, 0)
                for t in range(_K * H // 128):
                    sl = pl.ds(t * 128, 128)
                    cps.append(pltpu.async_copy(as_hbm.at[fidxr.at[t]], asg.at[sl], sem))
                    cps.append(pltpu.async_copy(ad_hbm.at[fidxc.at[t]], adg.at[sl], sem))
            elif first:
                for a in range(_NSUB):
                    sl = pl.ds(a * _SUB, _SUB)
                    cps.append(pltpu.async_copy(as_hbm.at[rowv2.at[a]], asg.at[sl], sem))
                    cps.append(pltpu.async_copy(ad_hbm.at[colg2.at[a]], adg.at[sl], sem))
            if kind == "num":
                for a in range(_NSUB):
                    sl = pl.ds(a * _SUB, _SUB)
                    cps.append(pltpu.async_copy(x_hbm.at[rowv2.at[a]], xrows.at[sl], sem))
            if not first:
                pltpu.sync_copy(w_hbm.at[pl.ds(s * EP * H + i * EPC, EPC)], wbuf)
            for cp in cps:
                cp.wait()

            wtgt = wv2 if kind == "num" else wexp
            if H == 8:

                def wloop(j, _, wtgt=wtgt, first=first):
                    if first:
                        av = asg[pl.ds(j * 16, 16)]
                        bv = adg[pl.ds(j * 16, 16)]
                        sv = av + bv
                        lr = jnp.maximum(sv, 0.2 * sv)
                        tv = g + bv
                        cb = jnp.maximum(tv, 0.2 * tv)
                        w = jnp.exp(lr - cb)
                        wbuf[pl.ds(j * 16, 16)] = w
                    else:
                        w = wbuf[pl.ds(j * 16, 16)]
                    we = _take16(w, low3)
                    wo = _take16(w, low3 + 8)
                    wtgt[2 * j, pl.ds(0, 16)] = jnp.where(lo8, we, 0.0)
                    wtgt[2 * j + 1, pl.ds(0, 16)] = jnp.where(lo8, wo, 0.0)
                    return 0

                lax.fori_loop(0, _K * H // 16, wloop, 0)
                if first:
                    pltpu.sync_copy(wbuf, w_hbm.at[pl.ds(s * EP * H + i * EPC, EPC)])
            elif first:

                def wloop(j, _):
                    av = asg[pl.ds(j * 16, 16)]
                    bv = adg[pl.ds(j * 16, 16)]
                    sv = av + bv
                    lr = jnp.maximum(sv, 0.2 * sv)
                    tv = g + bv
                    cb = jnp.maximum(tv, 0.2 * tv)
                    wbuf[pl.ds(j * 16, 16)] = jnp.exp(lr - cb)
                    return 0

                lax.fori_loop(0, _K // 16, wloop, 0)
                pltpu.sync_copy(wbuf, w_hbm.at[pl.ds(s * EP * H + i * EPC, EPC)])

            if kind == "num":
                if H == 8:

                    def mloop(k, _):
                        wrow = wv2[k, pl.ds(0, 16)]
                        for jp in range(Dh // 32):
                            ws = _take16(wrow, zl + (c * (Dh // 32) + jp))
                            xrows[k, pl.ds(jp * 32, 16)] = (
                                xrows[k, pl.ds(jp * 32, 16)] * ws)
                            xrows[k, pl.ds(jp * 32 + 16, 16)] = (
                                xrows[k, pl.ds(jp * 32 + 16, 16)] * ws)
                        return 0
                else:

                    def mloop(k, _):
                        st = pl.multiple_of((k >> 4) * 16, 16)
                        v = wbuf[pl.ds(st, 16)]
                        ws = _take16(v, zl + (k & 15))
                        for jp in range(C // 16):
                            xrows[k, pl.ds(jp * 16, 16)] = (
                                xrows[k, pl.ds(jp * 16, 16)] * ws)
                        return 0

                lax.fori_loop(0, _K, mloop, 0)
                for a in range(_NSUB):
                    sl = pl.ds(a * _SUB, _SUB)
                    pltpu.sync_copy(xrows.at[sl], acc.at[colv2.at[a]], add=True)
            else:
                if H == 1:
                    def dloop(k, _):
                        st = pl.multiple_of((k >> 4) * 16, 16)
                        v = wbuf[pl.ds(st, 16)]
                        ws = _take16(v, zl + (k & 15))
                        wexp[k, pl.ds(0, 16)] = jnp.where(lo8, ws, 0.0)
                        return 0

                    lax.fori_loop(0, _K, dloop, 0)
                for a in range(_NSUB):
                    sl = pl.ds(a * _SUB, _SUB)
                    pltpu.sync_copy(wexp.at[sl], acc.at[colv2.at[a]], add=True)
            return carry

        lax.fori_loop(0, nchunk, chunk, 0)
        plsc.subcore_barrier()

        def floop(j, _, kind=kind):
            o = s * RPF + j * 80
            if kind == "num":
                if H == 8:
                    pltpu.sync_copy(acc.at[pl.ds(o, 80)],
                                    num_out.at[c, pl.ds(off + o, 80)])
                else:
                    pltpu.sync_copy(acc.at[pl.ds(o, 80)],
                                    num_out.at[pl.ds(off + o, 80)])
            else:
                if H == 8:
                    @pl.when(c == 0)
                    def _():
                        pltpu.sync_copy(acc.at[pl.ds(o, 80)],
                                        den_out.at[pl.ds(off + o, 80)])
                else:
                    pltpu.sync_copy(acc.at[pl.ds(o, 80)],
                                    den_out.at[pl.ds(off + o, 80)])
            return 0

        lax.fori_loop(0, RPF // 80, floop, 0)
        plsc.subcore_barrier()


@functools.lru_cache(maxsize=None)
def _edge_kernel(n_src, np_dst, H, C):
    Dh = C // 2
    HALF = np_dst // 2
    mesh = plsc.VectorSubcoreMesh(core_axis_name="c", subcore_axis_name="s")
    if H == 8:
        out_type = [
            jax.ShapeDtypeStruct((2, np_dst, Dh), jnp.float32),
            jax.ShapeDtypeStruct((np_dst, 128), jnp.float32),
            jax.ShapeDtypeStruct((E * H,), jnp.float32),
        ]
        xw = Dh
    else:
        out_type = [
            jax.ShapeDtypeStruct((np_dst, C), jnp.float32),
            jax.ShapeDtypeStruct((np_dst, 128), jnp.float32),
            jax.ShapeDtypeStruct((E * H,), jnp.float32),
        ]
        xw = C
    return pl.kernel(
        functools.partial(_edge_body, n_src, np_dst, H, C),
        out_type=out_type,
        mesh=mesh,
        scratch_types=[
            pltpu.VMEM((_K,), jnp.int32),
            pltpu.VMEM((_K,), jnp.int32),
            pltpu.VMEM((_NSUB, _SUB), jnp.int32),
            pltpu.VMEM((_NSUB, _SUB), jnp.int32),
            pltpu.VMEM((_NSUB, _SUB), jnp.int32),
            pltpu.VMEM((_K * H // 128 if H == 8 else 1, 128), jnp.int32),
            pltpu.VMEM((_K * H // 128 if H == 8 else 1, 128), jnp.int32),
            pltpu.VMEM((_K * H,), jnp.float32),
            pltpu.VMEM((_K * H,), jnp.float32),
            pltpu.VMEM((_K * H,), jnp.float32),
            pltpu.VMEM((_K, 16), jnp.float32),
            pltpu.VMEM((_K, 128), jnp.float32),
            pltpu.VMEM((_K, xw), jnp.float32),
            pltpu.VMEM((16,), jnp.float32),
            pltpu.VMEM_SHARED((HALF + 128, 128), jnp.float32),
            pltpu.SemaphoreType.DMA,
        ],
    )


def _edge_stage(xtab, a_src, a_dst, gvec, row, col, n_src, n_dst, H, C):
    Dh = C // 2
    np_dst = -(-n_dst // 256) * 256
    HALF = np_dst // 2
    k = _edge_kernel(n_src, np_dst, H, C)
    za = jnp.zeros((HALF + 128, 128), jnp.float32)
    num, den, _w = k(row, col, xtab, a_src.reshape(-1), a_dst.reshape(-1),
                     gvec, za)
    if H == 8:
        return num[0, :n_dst], num[1, :n_dst], den[:n_dst, :H]
    return num[:n_dst, :Dh], num[:n_dst, Dh:], den[:n_dst, :H]


def _ls_mat(ls, H, D):
    return (ls[0][:, :, None] * jnp.eye(H, dtype=jnp.float32)[:, None, :]).reshape(H * D, H)


def kernel(x_bug, x_dev, ei_bug_to_dev, ei_dev_to_bug, ei_bug_dup_bug, params):
    p1 = params["han1"]
    p2 = params["han2"]

    lsb = jnp.concatenate([
        _ls_mat(p1["lin_src"]["bug__to__dev"], 8, 32),
        _ls_mat(p1["lin_dst"]["dev__to__bug"], 8, 32),
        _ls_mat(p1["lin_src"]["bug__dup__bug"], 8, 32),
        _ls_mat(p1["lin_dst"]["bug__dup__bug"], 8, 32),
    ], axis=1)
    lsd = jnp.concatenate([
        _ls_mat(p1["lin_dst"]["bug__to__dev"], 8, 32),
        _ls_mat(p1["lin_src"]["dev__to__bug"], 8, 32),
    ], axis=1)
    xb, xd, ab, ad, gb, gd = _proj1(
        x_bug, x_dev,
        p1["proj"]["bug"]["W"], p1["proj"]["bug"]["b"].reshape(1, 256),
        p1["proj"]["dev"]["W"], p1["proj"]["dev"]["b"].reshape(1, 256),
        lsb, lsd)

    xb_both = jnp.concatenate([xb[:, :128], xb[:, 128:]], axis=0)
    xd_both = jnp.concatenate([xd[:, :128], xd[:, 128:]], axis=0)

    def gv8(g):
        return jnp.tile(g.reshape(-1), 2)

    n_b2d = _edge_stage(xb_both, ab[:, 0:8], ad[:, 0:8], gv8(gb[:, 0:8]),
                        ei_bug_to_dev[0], ei_bug_to_dev[1], N_NODE, N_NODE, 8, 256)
    n_d2b = _edge_stage(xd_both, ad[:, 8:16], ab[:, 8:16], gv8(gd[:, 8:16]),
                        ei_dev_to_bug[0], ei_dev_to_bug[1], N_NODE, N_NODE, 8, 256)
    n_dup = _edge_stage(xb_both, ab[:, 16:24], ab[:, 24:32], gv8(gb[:, 16:24]),
                        ei_bug_dup_bug[0], ei_bug_dup_bug[1], N_NODE, N_NODE, 8, 256)

    stb, tb = _norm_t([(n_d2b[0], n_d2b[1]), (n_dup[0], n_dup[1])],
                      [n_d2b[2], n_dup[2]],
                      p1["k_lin"]["W"], p1["k_lin"]["b"].reshape(1, 256), 8, 256)
    std, td = _norm_t([(n_b2d[0], n_b2d[1])], [n_b2d[2]],
                      p1["k_lin"]["W"], p1["k_lin"]["b"].reshape(1, 256), 8, 256)

    lsb2 = jnp.concatenate([
        _ls_mat(p2["lin_src"]["bug__dup__bug"], 1, 128),
        _ls_mat(p2["lin_dst"]["bug__dup__bug"], 1, 128),
        _ls_mat(p2["lin_dst"]["dev__to__bug"], 1, 128),
    ], axis=1)
    lsd2 = _ls_mat(p2["lin_src"]["dev__to__bug"], 1, 128)
    xb2, xd2, ab2, ad2, gb2, gd2 = _mix2(
        stb, tb, p1["q"], std[0], td,
        p2["proj"]["bug"]["W"], p2["proj"]["bug"]["b"].reshape(1, 128),
        p2["proj"]["dev"]["W"], p2["proj"]["dev"]["b"].reshape(1, 128),
        lsb2, lsd2)

    g_d2b2 = jnp.tile(gd2.reshape(1), 16)
    g_dup2 = jnp.tile(gb2[:, 0].reshape(1), 16)

    n2_d2b = _edge_stage(xd2, ad2[:, 0:1], ab2[:, 2:3], g_d2b2,
                         ei_dev_to_bug[0], ei_dev_to_bug[1], N_NODE, N_NODE, 1, 128)
    n2_dup = _edge_stage(xb2, ab2[:, 0:1], ab2[:, 1:2], g_dup2,
                         ei_bug_dup_bug[0], ei_bug_dup_bug[1], N_NODE, N_NODE, 1, 128)

    st2, t2 = _norm_t([(n2_d2b[0], n2_d2b[1]), (n2_dup[0], n2_dup[1])],
                      [n2_d2b[2], n2_dup[2]],
                      p2["k_lin"]["W"], p2["k_lin"]["b"].reshape(1, 128), 1, 128)

    return _final(st2, t2, p2["q"], params["cls"]["W"],
                  params["cls"]["b"].reshape(1, 10))


# speedup vs baseline: 12.3985x; 1.1385x over previous
import functools

import jax
import jax.numpy as jnp
from jax import lax
from jax.experimental import pallas as pl
from jax.experimental.pallas import tpu as pltpu
from jax.experimental.pallas import tpu_sc as plsc

N_NODE = 10000
E = 160000
BM = 400
GRID_M = N_NODE // BM


def _proj1_body(xb_ref, xd_ref, wb_ref, bb_ref, wd_ref, bd_ref, lsb_ref, lsd_ref,
                xbo_ref, xdo_ref, ab_ref, ad_ref, gb_ref, gd_ref):
    i = pl.program_id(0)
    xb = jnp.dot(xb_ref[...], wb_ref[...], preferred_element_type=jnp.float32) + bb_ref[...]
    xd = jnp.dot(xd_ref[...], wd_ref[...], preferred_element_type=jnp.float32) + bd_ref[...]
    xbo_ref[...] = xb
    xdo_ref[...] = xd
    ab = jnp.dot(xb, lsb_ref[...], preferred_element_type=jnp.float32)
    ad = jnp.dot(xd, lsd_ref[...], preferred_element_type=jnp.float32)
    ab_ref[...] = ab
    ad_ref[...] = ad
    gb = jnp.max(ab, axis=0, keepdims=True)
    gd = jnp.max(ad, axis=0, keepdims=True)

    @pl.when(i == 0)
    def _():
        gb_ref[...] = gb
        gd_ref[...] = gd

    @pl.when(i != 0)
    def _():
        gb_ref[...] = jnp.maximum(gb_ref[...], gb)
        gd_ref[...] = jnp.maximum(gd_ref[...], gd)


def _proj1(x_bug, x_dev, wb, bb, wd, bd, lsb, lsd):
    return pl.pallas_call(
        _proj1_body,
        grid=(GRID_M,),
        in_specs=[
            pl.BlockSpec((BM, 256), lambda i: (i, 0)),
            pl.BlockSpec((BM, 256), lambda i: (i, 0)),
            pl.BlockSpec((256, 256), lambda i: (0, 0)),
            pl.BlockSpec((1, 256), lambda i: (0, 0)),
            pl.BlockSpec((256, 256), lambda i: (0, 0)),
            pl.BlockSpec((1, 256), lambda i: (0, 0)),
            pl.BlockSpec((256, 32), lambda i: (0, 0)),
            pl.BlockSpec((256, 16), lambda i: (0, 0)),
        ],
        out_specs=[
            pl.BlockSpec((BM, 256), lambda i: (i, 0)),
            pl.BlockSpec((BM, 256), lambda i: (i, 0)),
            pl.BlockSpec((BM, 32), lambda i: (i, 0)),
            pl.BlockSpec((BM, 16), lambda i: (i, 0)),
            pl.BlockSpec((1, 32), lambda i: (0, 0)),
            pl.BlockSpec((1, 16), lambda i: (0, 0)),
        ],
        out_shape=[
            jax.ShapeDtypeStruct((N_NODE, 256), jnp.float32),
            jax.ShapeDtypeStruct((N_NODE, 256), jnp.float32),
            jax.ShapeDtypeStruct((N_NODE, 32), jnp.float32),
            jax.ShapeDtypeStruct((N_NODE, 16), jnp.float32),
            jax.ShapeDtypeStruct((1, 32), jnp.float32),
            jax.ShapeDtypeStruct((1, 16), jnp.float32),
        ],
    )(x_bug, x_dev, wb, bb, wd, bd, lsb, lsd)


def _norm_t_body(H, C, R, *refs):
    i = pl.program_id(0)
    ins = refs[:3 * R + 2]
    outs = refs[3 * R + 2:]
    kw_ref, kb_ref = ins[3 * R], ins[3 * R + 1]
    t_ref = outs[R]
    D = C // H
    for r in range(R):
        n0 = ins[3 * r][...]
        n1 = ins[3 * r + 1][...]
        den = ins[3 * r + 2][...]
        num = jnp.concatenate([n0, n1], axis=1)
        if H > 1:
            num3 = num.reshape(-1, H, D)
            st3 = num3 / (den[:, :, None] + 1e-16)
            st = jnp.maximum(st3.reshape(-1, C), 0.0)
        else:
            st = jnp.maximum(num / (den + 1e-16), 0.0)
        outs[r][...] = st
        tt = jnp.tanh(jnp.dot(st, kw_ref[...], preferred_element_type=jnp.float32)
                      + kb_ref[...])
        tsum = jnp.sum(tt, axis=0, keepdims=True)

        @pl.when(i == 0)
        def _(r=r, tsum=tsum):
            t_ref[r, :] = tsum[0]

        @pl.when(i != 0)
        def _(r=r, tsum=tsum):
            t_ref[r, :] = t_ref[r, :] + tsum[0]


def _norm_t(nums, dens, kw, kb, H, C):
    R = len(nums)
    Dh = C // 2
    in_specs = []
    args = []
    for (n0, n1), den in zip(nums, dens):
        in_specs += [pl.BlockSpec((BM, Dh), lambda i: (i, 0)),
                     pl.BlockSpec((BM, Dh), lambda i: (i, 0)),
                     pl.BlockSpec((BM, H), lambda i: (i, 0))]
        args += [n0, n1, den]
    in_specs += [pl.BlockSpec((C, C), lambda i: (0, 0)),
                 pl.BlockSpec((1, C), lambda i: (0, 0))]
    args += [kw, kb]
    out_specs = [pl.BlockSpec((BM, C), lambda i: (i, 0)) for _ in range(R)]
    out_specs += [pl.BlockSpec((R, C), lambda i: (0, 0))]
    out_shape = [jax.ShapeDtypeStruct((N_NODE, C), jnp.float32) for _ in range(R)]
    out_shape += [jax.ShapeDtypeStruct((R, C), jnp.float32)]
    res = pl.pallas_call(
        functools.partial(_norm_t_body, H, C, R),
        grid=(GRID_M,),
        in_specs=in_specs,
        out_specs=out_specs,
        out_shape=out_shape,
    )(*args)
    return list(res[:R]), res[R]


def _mix2_body(stb0_ref, stb1_ref, tb_ref, q1_ref,
               std_ref, td_ref,
               wb2_ref, bb2_ref, wd2_ref, bd2_ref, lsb2_ref, lsd2_ref,
               xb2_ref, xd2_ref, ab2_ref, ad2_ref, gb2_ref, gd2_ref):
    i = pl.program_id(0)
    q = q1_ref[...]
    tb = tb_ref[...] * (1.0 / N_NODE)
    s0 = jnp.sum(q[0] * tb[0])
    s1 = jnp.sum(q[0] * tb[1])
    m = jnp.maximum(s0, s1)
    e0 = jnp.exp(s0 - m)
    e1 = jnp.exp(s1 - m)
    inv = 1.0 / (e0 + e1)
    hb = stb0_ref[...] * (e0 * inv) + stb1_ref[...] * (e1 * inv)
    hb = jnp.where(hb > 0, hb, jnp.exp(jnp.minimum(hb, 0.0)) - 1.0)
    hd = std_ref[...]
    hd = jnp.where(hd > 0, hd, jnp.exp(jnp.minimum(hd, 0.0)) - 1.0)
    del td_ref
    xb2 = jnp.dot(hb, wb2_ref[...], preferred_element_type=jnp.float32) + bb2_ref[...]
    xd2 = jnp.dot(hd, wd2_ref[...], preferred_element_type=jnp.float32) + bd2_ref[...]
    xb2_ref[...] = xb2
    xd2_ref[...] = xd2
    ab2 = jnp.dot(xb2, lsb2_ref[...], preferred_element_type=jnp.float32)
    ad2 = jnp.dot(xd2, lsd2_ref[...], preferred_element_type=jnp.float32)
    ab2_ref[...] = ab2
    ad2_ref[...] = ad2
    gb2 = jnp.max(ab2, axis=0, keepdims=True)
    gd2 = jnp.max(ad2, axis=0, keepdims=True)

    @pl.when(i == 0)
    def _():
        gb2_ref[...] = gb2
        gd2_ref[...] = gd2

    @pl.when(i != 0)
    def _():
        gb2_ref[...] = jnp.maximum(gb2_ref[...], gb2)
        gd2_ref[...] = jnp.maximum(gd2_ref[...], gd2)


def _mix2(stb, tb, q1, std, td, wb2, bb2, wd2, bd2, lsb2, lsd2):
    return pl.pallas_call(
        _mix2_body,
        grid=(GRID_M,),
        in_specs=[
            pl.BlockSpec((BM, 256), lambda i: (i, 0)),
            pl.BlockSpec((BM, 256), lambda i: (i, 0)),
            pl.BlockSpec((2, 256), lambda i: (0, 0)),
            pl.BlockSpec((1, 256), lambda i: (0, 0)),
            pl.BlockSpec((BM, 256), lambda i: (i, 0)),
            pl.BlockSpec((1, 256), lambda i: (0, 0)),
            pl.BlockSpec((256, 128), lambda i: (0, 0)),
            pl.BlockSpec((1, 128), lambda i: (0, 0)),
            pl.BlockSpec((256, 128), lambda i: (0, 0)),
            pl.BlockSpec((1, 128), lambda i: (0, 0)),
            pl.BlockSpec((128, 3), lambda i: (0, 0)),
            pl.BlockSpec((128, 1), lambda i: (0, 0)),
        ],
        out_specs=[
            pl.BlockSpec((BM, 128), lambda i: (i, 0)),
            pl.BlockSpec((BM, 128), lambda i: (i, 0)),
            pl.BlockSpec((BM, 3), lambda i: (i, 0)),
            pl.BlockSpec((BM, 1), lambda i: (i, 0)),
            pl.BlockSpec((1, 3), lambda i: (0, 0)),
            pl.BlockSpec((1, 1), lambda i: (0, 0)),
        ],
        out_shape=[
            jax.ShapeDtypeStruct((N_NODE, 128), jnp.float32),
            jax.ShapeDtypeStruct((N_NODE, 128), jnp.float32),
            jax.ShapeDtypeStruct((N_NODE, 3), jnp.float32),
            jax.ShapeDtypeStruct((N_NODE, 1), jnp.float32),
            jax.ShapeDtypeStruct((1, 3), jnp.float32),
            jax.ShapeDtypeStruct((1, 1), jnp.float32),
        ],
    )(stb[0], stb[1], tb, q1, std, td, wb2, bb2, wd2, bd2, lsb2, lsd2)


def _final_body(st0_ref, st1_ref, t_ref, q2_ref, cw_ref, cb_ref, out_ref):
    q = q2_ref[...]
    t = t_ref[...] * (1.0 / N_NODE)
    s0 = jnp.sum(q[0] * t[0])
    s1 = jnp.sum(q[0] * t[1])
    m = jnp.maximum(s0, s1)
    e0 = jnp.exp(s0 - m)
    e1 = jnp.exp(s1 - m)
    inv = 1.0 / (e0 + e1)
    h = st0_ref[...] * (e0 * inv) + st1_ref[...] * (e1 * inv)
    out_ref[...] = (jnp.dot(h, cw_ref[...], preferred_element_type=jnp.float32)
                    + cb_ref[...])


def _final(st, t, q2, cw, cb):
    return pl.pallas_call(
        _final_body,
        grid=(GRID_M,),
        in_specs=[
            pl.BlockSpec((BM, 128), lambda i: (i, 0)),
            pl.BlockSpec((BM, 128), lambda i: (i, 0)),
            pl.BlockSpec((2, 128), lambda i: (0, 0)),
            pl.BlockSpec((1, 128), lambda i: (0, 0)),
            pl.BlockSpec((128, 10), lambda i: (0, 0)),
            pl.BlockSpec((1, 10), lambda i: (0, 0)),
        ],
        out_specs=[pl.BlockSpec((BM, 10), lambda i: (i, 0))],
        out_shape=[jax.ShapeDtypeStruct((N_NODE, 10), jnp.float32)],
    )(st[0], st[1], t, q2, cw, cb)[0]


_K = 80
_SUB = 80
_NSUB = _K // _SUB


def _take16(v, idx):
    return lax.gather(
        v, idx[:, None],
        lax.GatherDimensionNumbers(offset_dims=(), collapsed_slice_dims=(0,),
                                   start_index_map=(0,)),
        (1,), mode=lax.GatherScatterMode.PROMISE_IN_BOUNDS)


def _edge_body(n_src, np_dst, H, C, row_hbm, col_hbm, x_hbm, as_hbm, ad_hbm,
               gv_hbm, za_hbm, num_out, den_out, w_hbm,
               rowf, colf, colv2, colg2, rowv2, fidxr, fidxc, asg, adg, wbuf,
               wv2, wexp, xrows, gvv, acc, sem):
    Dh = C // 2
    HALF = np_dst // 2
    c = lax.axis_index("c")
    s = lax.axis_index("s")
    EP = E // 16
    nchunk = EP // _K
    RPZ = (HALF + 128) // 16
    RPF = HALF // 16

    pltpu.sync_copy(gv_hbm, gvv)
    pltpu.sync_copy(za_hbm.at[pl.ds(0, _K)], wexp)
    g = gvv[...]
    cn = c * n_src if H == 8 else 0
    il = lax.iota(jnp.int32, 16)
    zl = il & 0
    lo8 = il < 8
    low3 = il & 7
    pair = il >> 3

    if H == 8:
        passes = [("num", 0), ("num", 1), ("den", None)]
    else:
        passes = [("num", None), ("den", None)]

    EPC = _K * H

    for pi, (kind, ph) in enumerate(passes):
        first = pi == 0
        off = ph * HALF if ph is not None else c * HALF

        def iloop(j, _):
            o = s * RPZ + j * 80
            pltpu.sync_copy(za_hbm.at[pl.ds(o, 80)], acc.at[pl.ds(o, 80)])
            return 0

        lax.fori_loop(0, RPZ // 80, iloop, 0)
        o8 = s * RPZ + (RPZ // 80) * 80
        pltpu.sync_copy(za_hbm.at[pl.ds(o8, RPZ % 80)], acc.at[pl.ds(o8, RPZ % 80)])
        plsc.subcore_barrier()

        def chunk(i, carry, kind=kind):
            base = s * EP + i * _K
            pltpu.sync_copy(row_hbm.at[pl.ds(base, _K)], rowf)
            pltpu.sync_copy(col_hbm.at[pl.ds(base, _K)], colf)
            for b in range(_K // 16):
                cv = colf[pl.ds(b * 16, 16)]
                sh = cv - off
                ok = (sh >= 0) & (sh < HALF)
                a0, b0 = b // (_SUB // 16), (b % (_SUB // 16)) * 16
                colv2[a0, pl.ds(b0, 16)] = jnp.where(ok, sh, HALF)
                colg2[a0, pl.ds(b0, 16)] = cv
                rowv2[a0, pl.ds(b0, 16)] = rowf[pl.ds(b * 16, 16)] + cn

            cps = []
            if H == 8 and first:
                def eloop(jb, _):
                    rv = rowf[pl.ds(jb * 16, 16)]
                    cv = colf[pl.ds(jb * 16, 16)]
                    for sub in range(8):
                        idxr = _take16(rv, pair + 2 * sub)
                        idxc = _take16(cv, pair + 2 * sub)
                        fidxr[jb, pl.ds(sub * 16, 16)] = idxr * 8 + low3
                        fidxc[jb, pl.ds(sub * 16, 16)] = idxc * 8 + low3
                    return 0

                lax.fori_loop(0, _K // 16, eloop, 0)
                for t in range(_K * H // 128):
                    sl = pl.ds(t * 128, 128)
                    cps.append(pltpu.async_copy(as_hbm.at[fidxr.at[t]], asg.at[sl], sem))
                    cps.append(pltpu.async_copy(ad_hbm.at[fidxc.at[t]], adg.at[sl], sem))
            elif first:
                for a in range(_NSUB):
                    sl = pl.ds(a * _SUB, _SUB)
                    cps.append(pltpu.async_copy(as_hbm.at[rowv2.at[a]], asg.at[sl], sem))
                    cps.append(pltpu.async_copy(ad_hbm.at[colg2.at[a]], adg.at[sl], sem))
            if kind == "num":
                for a in range(_NSUB):
                    sl = pl.ds(a * _SUB, _SUB)
                    cps.append(pltpu.async_copy(x_hbm.at[rowv2.at[a]], xrows.at[sl], sem))
            if not first:
                pltpu.sync_copy(w_hbm.at[pl.ds(s * EP * H + i * EPC, EPC)], wbuf)
            for cp in cps:
                cp.wait()

            wtgt = wv2 if kind == "num" else wexp
            if H == 8:

                def wloop(j, _, wtgt=wtgt, first=first):
                    if first:
                        av = asg[pl.ds(j * 16, 16)]
                        bv = adg[pl.ds(j * 16, 16)]
                        sv = av + bv
                        lr = jnp.maximum(sv, 0.2 * sv)
                        tv = g + bv
                        cb = jnp.maximum(tv, 0.2 * tv)
                        w = jnp.exp(lr - cb)
                        wbuf[pl.ds(j * 16, 16)] = w
                    else:
                        w = wbuf[pl.ds(j * 16, 16)]
                    we = _take16(w, low3)
                    wo = _take16(w, low3 + 8)
                    wtgt[2 * j, pl.ds(0, 16)] = jnp.where(lo8, we, 0.0)
                    wtgt[2 * j + 1, pl.ds(0, 16)] = jnp.where(lo8, wo, 0.0)
                    return 0

                lax.fori_loop(0, _K * H // 16, wloop, 0)
                if first:
                    pltpu.sync_copy(wbuf, w_hbm.at[pl.ds(s * EP * H + i * EPC, EPC)])
            elif first:

                def wloop(j, _):
                    av = asg[pl.ds(j * 16, 16)]
                    bv = adg[pl.ds(j * 16, 16)]
                    sv = av + bv
                    lr = jnp.maximum(sv, 0.2 * sv)
                    tv = g + bv
                    cb = jnp.maximum(tv, 0.2 * tv)
                    wbuf[pl.ds(j * 16, 16)] = jnp.exp(lr - cb)
                    return 0

                lax.fori_loop(0, _K // 16, wloop, 0)
                pltpu.sync_copy(wbuf, w_hbm.at[pl.ds(s * EP * H + i * EPC, EPC)])

            if kind == "num":
                if H == 8:

                    def mloop(k, _):
                        wrow = wv2[k, pl.ds(0, 16)]
                        for jp in range(Dh // 32):
                            ws = _take16(wrow, zl + (c * (Dh // 32) + jp))
                            xrows[k, pl.ds(jp * 32, 16)] = (
                                xrows[k, pl.ds(jp * 32, 16)] * ws)
                            xrows[k, pl.ds(jp * 32 + 16, 16)] = (
                                xrows[k, pl.ds(jp * 32 + 16, 16)] * ws)
                        return 0
                else:

                    def mloop(k, _):
                        st = pl.multiple_of((k >> 4) * 16, 16)
                        v = wbuf[pl.ds(st, 16)]
                        ws = _take16(v, zl + (k & 15))
                        for jp in range(C // 16):
                            xrows[k, pl.ds(jp * 16, 16)] = (
                                xrows[k, pl.ds(jp * 16, 16)] * ws)
                        return 0

                lax.fori_loop(0, _K, mloop, 0)
                for a in range(_NSUB):
                    sl = pl.ds(a * _SUB, _SUB)
                    pltpu.sync_copy(xrows.at[sl], acc.at[colv2.at[a]], add=True)
            else:
                if H == 1:
                    def dloop(k, _):
                        st = pl.multiple_of((k >> 4) * 16, 16)
                        v = wbuf[pl.ds(st, 16)]
                        ws = _take16(v, zl + (k & 15))
                        wexp[k, pl.ds(0, 16)] = jnp.where(lo8, ws, 0.0)
                        return 0

                    lax.fori_loop(0, _K, dloop, 0)
                for a in range(_NSUB):
                    sl = pl.ds(a * _SUB, _SUB)
                    pltpu.sync_copy(wexp.at[sl], acc.at[colv2.at[a]], add=True)
            return carry

        lax.fori_loop(0, nchunk, chunk, 0)
        plsc.subcore_barrier()

        def floop(j, _, kind=kind):
            o = s * RPF + j * 80
            if kind == "num":
                if H == 8:
                    pltpu.sync_copy(acc.at[pl.ds(o, 80)],
                                    num_out.at[c, pl.ds(off + o, 80)])
                else:
                    pltpu.sync_copy(acc.at[pl.ds(o, 80)],
                                    num_out.at[pl.ds(off + o, 80)])
            else:
                pltpu.sync_copy(acc.at[pl.ds(o, 80)],
                                den_out.at[pl.ds(off + o, 80)])
            return 0

        lax.fori_loop(0, RPF // 80, floop, 0)
        plsc.subcore_barrier()


@functools.lru_cache(maxsize=None)
def _edge_kernel(n_src, np_dst, H, C):
    Dh = C // 2
    HALF = np_dst // 2
    mesh = plsc.VectorSubcoreMesh(core_axis_name="c", subcore_axis_name="s")
    if H == 8:
        out_type = [
            jax.ShapeDtypeStruct((2, np_dst, Dh), jnp.float32),
            jax.ShapeDtypeStruct((np_dst, 128), jnp.float32),
            jax.ShapeDtypeStruct((E * H,), jnp.float32),
        ]
        xw = Dh
    else:
        out_type = [
            jax.ShapeDtypeStruct((np_dst, C), jnp.float32),
            jax.ShapeDtypeStruct((np_dst, 128), jnp.float32),
            jax.ShapeDtypeStruct((E * H,), jnp.float32),
        ]
        xw = C
    return pl.kernel(
        functools.partial(_edge_body, n_src, np_dst, H, C),
        out_type=out_type,
        mesh=mesh,
        scratch_types=[
            pltpu.VMEM((_K,), jnp.int32),
            pltpu.VMEM((_K,), jnp.int32),
            pltpu.VMEM((_NSUB, _SUB), jnp.int32),
            pltpu.VMEM((_NSUB, _SUB), jnp.int32),
            pltpu.VMEM((_NSUB, _SUB), jnp.int32),
            pltpu.VMEM((_K * H // 128 if H == 8 else 1, 128), jnp.int32),
            pltpu.VMEM((_K * H // 128 if H == 8 else 1, 128), jnp.int32),
            pltpu.VMEM((_K * H,), jnp.float32),
            pltpu.VMEM((_K * H,), jnp.float32),
            pltpu.VMEM((_K * H,), jnp.float32),
            pltpu.VMEM((_K, 16), jnp.float32),
            pltpu.VMEM((_K, 128), jnp.float32),
            pltpu.VMEM((_K, xw), jnp.float32),
            pltpu.VMEM((16,), jnp.float32),
            pltpu.VMEM_SHARED((HALF + 128, 128), jnp.float32),
            pltpu.SemaphoreType.DMA,
        ],
    )


def _edge_stage(xtab, a_src, a_dst, gvec, row, col, n_src, n_dst, H, C):
    Dh = C // 2
    np_dst = -(-n_dst // 256) * 256
    HALF = np_dst // 2
    k = _edge_kernel(n_src, np_dst, H, C)
    za = jnp.zeros((HALF + 128, 128), jnp.float32)
    num, den, _w = k(row, col, xtab, a_src.reshape(-1), a_dst.reshape(-1),
                     gvec, za)
    if H == 8:
        return num[0, :n_dst], num[1, :n_dst], den[:n_dst, :H]
    return num[:n_dst, :Dh], num[:n_dst, Dh:], den[:n_dst, :H]


def _ls_mat(ls, H, D):
    return (ls[0][:, :, None] * jnp.eye(H, dtype=jnp.float32)[:, None, :]).reshape(H * D, H)


def kernel(x_bug, x_dev, ei_bug_to_dev, ei_dev_to_bug, ei_bug_dup_bug, params):
    p1 = params["han1"]
    p2 = params["han2"]

    lsb = jnp.concatenate([
        _ls_mat(p1["lin_src"]["bug__to__dev"], 8, 32),
        _ls_mat(p1["lin_dst"]["dev__to__bug"], 8, 32),
        _ls_mat(p1["lin_src"]["bug__dup__bug"], 8, 32),
        _ls_mat(p1["lin_dst"]["bug__dup__bug"], 8, 32),
    ], axis=1)
    lsd = jnp.concatenate([
        _ls_mat(p1["lin_dst"]["bug__to__dev"], 8, 32),
        _ls_mat(p1["lin_src"]["dev__to__bug"], 8, 32),
    ], axis=1)
    xb, xd, ab, ad, gb, gd = _proj1(
        x_bug, x_dev,
        p1["proj"]["bug"]["W"], p1["proj"]["bug"]["b"].reshape(1, 256),
        p1["proj"]["dev"]["W"], p1["proj"]["dev"]["b"].reshape(1, 256),
        lsb, lsd)

    xb_both = jnp.concatenate([xb[:, :128], xb[:, 128:]], axis=0)
    xd_both = jnp.concatenate([xd[:, :128], xd[:, 128:]], axis=0)

    def gv8(g):
        return jnp.tile(g.reshape(-1), 2)

    n_b2d = _edge_stage(xb_both, ab[:, 0:8], ad[:, 0:8], gv8(gb[:, 0:8]),
                        ei_bug_to_dev[0], ei_bug_to_dev[1], N_NODE, N_NODE, 8, 256)
    n_d2b = _edge_stage(xd_both, ad[:, 8:16], ab[:, 8:16], gv8(gd[:, 8:16]),
                        ei_dev_to_bug[0], ei_dev_to_bug[1], N_NODE, N_NODE, 8, 256)
    n_dup = _edge_stage(xb_both, ab[:, 16:24], ab[:, 24:32], gv8(gb[:, 16:24]),
                        ei_bug_dup_bug[0], ei_bug_dup_bug[1], N_NODE, N_NODE, 8, 256)

    stb, tb = _norm_t([(n_d2b[0], n_d2b[1]), (n_dup[0], n_dup[1])],
                      [n_d2b[2], n_dup[2]],
                      p1["k_lin"]["W"], p1["k_lin"]["b"].reshape(1, 256), 8, 256)
    std, td = _norm_t([(n_b2d[0], n_b2d[1])], [n_b2d[2]],
                      p1["k_lin"]["W"], p1["k_lin"]["b"].reshape(1, 256), 8, 256)

    lsb2 = jnp.concatenate([
        _ls_mat(p2["lin_src"]["bug__dup__bug"], 1, 128),
        _ls_mat(p2["lin_dst"]["bug__dup__bug"], 1, 128),
        _ls_mat(p2["lin_dst"]["dev__to__bug"], 1, 128),
    ], axis=1)
    lsd2 = _ls_mat(p2["lin_src"]["dev__to__bug"], 1, 128)
    xb2, xd2, ab2, ad2, gb2, gd2 = _mix2(
        stb, tb, p1["q"], std[0], td,
        p2["proj"]["bug"]["W"], p2["proj"]["bug"]["b"].reshape(1, 128),
        p2["proj"]["dev"]["W"], p2["proj"]["dev"]["b"].reshape(1, 128),
        lsb2, lsd2)

    g_d2b2 = jnp.tile(gd2.reshape(1), 16)
    g_dup2 = jnp.tile(gb2[:, 0].reshape(1), 16)

    n2_d2b = _edge_stage(xd2, ad2[:, 0:1], ab2[:, 2:3], g_d2b2,
                         ei_dev_to_bug[0], ei_dev_to_bug[1], N_NODE, N_NODE, 1, 128)
    n2_dup = _edge_stage(xb2, ab2[:, 0:1], ab2[:, 1:2], g_dup2,
                         ei_bug_dup_bug[0], ei_bug_dup_bug[1], N_NODE, N_NODE, 1, 128)

    st2, t2 = _norm_t([(n2_d2b[0], n2_d2b[1]), (n2_dup[0], n2_dup[1])],
                      [n2_d2b[2], n2_dup[2]],
                      p2["k_lin"]["W"], p2["k_lin"]["b"].reshape(1, 128), 1, 128)

    return _final(st2, t2, p2["q"], params["cls"]["W"],
                  params["cls"]["b"].reshape(1, 10))
